# Initial kernel scaffold; baseline (speedup 1.0000x reference)
import jax
import jax.numpy as jnp
from jax.experimental import pallas as pl


def kernel(node_features, edge_index, W1, b1, W2, b2):
    raise NotImplementedError("write your pallas kernel here")


# speedup vs baseline: 14.8693x; 14.8693x over previous
import functools
import jax
import jax.numpy as jnp
from jax import lax
from jax.experimental import pallas as pl
from jax.experimental.pallas import tpu as pltpu
from jax.experimental.pallas import tpu_sc as plsc

N_NODES = 10000
N_EDGES = 320000
D_IN = 128
D_HID = 128
N_CLS = 16

NUM_CORES = 2
NUM_SUBCORES = 16
NUM_WORKERS = NUM_CORES * NUM_SUBCORES
EDGES_PER_CORE = N_EDGES // NUM_CORES
EDGES_PER_WORKER = EDGES_PER_CORE // NUM_SUBCORES
EDGE_BATCH = 80
BATCHES_PER_WORKER = EDGES_PER_WORKER // EDGE_BATCH

_SC_MESH = plsc.VectorSubcoreMesh(
    core_axis_name="c", subcore_axis_name="s",
    num_cores=NUM_CORES, num_subcores=NUM_SUBCORES)


def _sc_degree(dst_hbm, zeros_hbm, ones_hbm, deg_out, deg_sh, idx_v, ones_v,
               gsem):
    cid = lax.axis_index("c")
    sid = lax.axis_index("s")

    @pl.when(sid == 0)
    def _():
        pltpu.sync_copy(zeros_hbm, deg_sh)
    pltpu.sync_copy(ones_hbm, ones_v)
    plsc.subcore_barrier()

    base0 = (cid * NUM_SUBCORES + sid) * EDGES_PER_WORKER

    def body(i, _):
        base = base0 + i * EDGE_BATCH
        pltpu.sync_copy(dst_hbm.at[pl.ds(base, EDGE_BATCH)], idx_v)
        pltpu.sync_copy(ones_v, deg_sh.at[idx_v], add=True)
        return 0

    lax.fori_loop(0, BATCHES_PER_WORKER, body, 0)
    plsc.subcore_barrier()

    @pl.when(sid == 0)
    def _():
        pltpu.sync_copy(deg_sh, deg_out.at[cid])


DEG_W = 16


def _degree_parts(dst, zeros_nw, ones_bw):
    return pl.kernel(
        _sc_degree,
        out_type=jax.ShapeDtypeStruct((NUM_CORES, N_NODES, DEG_W), jnp.float32),
        mesh=_SC_MESH,
        scratch_types=[
            pltpu.VMEM_SHARED((N_NODES, DEG_W), jnp.float32),
            pltpu.VMEM((EDGE_BATCH,), jnp.int32),
            pltpu.VMEM((EDGE_BATCH, DEG_W), jnp.float32),
            pltpu.SemaphoreType.DMA,
        ],
        compiler_params=pltpu.CompilerParams(use_tc_tiling_on_sc=False),
    )(dst, zeros_nw, ones_bw)


def _sc_aggregate(d, tab_hbm, src_hbm, dst_hbm, zeros_hbm, acc_out,
                  acc_sh, isrc_v, idst_v, rows_v, gsem):
    cid = lax.axis_index("c")
    sid = lax.axis_index("s")

    @pl.when(sid == 0)
    def _():
        pltpu.sync_copy(zeros_hbm, acc_sh)
    plsc.subcore_barrier()

    base0 = (cid * NUM_SUBCORES + sid) * EDGES_PER_WORKER

    def body(i, _):
        base = base0 + i * EDGE_BATCH
        pltpu.sync_copy(src_hbm.at[pl.ds(base, EDGE_BATCH)], isrc_v)
        pltpu.sync_copy(dst_hbm.at[pl.ds(base, EDGE_BATCH)], idst_v)
        pltpu.async_copy(tab_hbm.at[isrc_v], rows_v, gsem).wait()
        pltpu.sync_copy(rows_v, acc_sh.at[idst_v], add=True)
        return 0

    lax.fori_loop(0, BATCHES_PER_WORKER, body, 0)
    plsc.subcore_barrier()

    chunk = 624
    off = pl.multiple_of(sid * chunk, 8)
    pltpu.sync_copy(
        acc_sh.at[pl.ds(off, chunk)],
        acc_out.at[cid].at[pl.ds(off, chunk)],
    )

    @pl.when(sid == NUM_SUBCORES - 1)
    def _():
        tail = N_NODES - NUM_SUBCORES * chunk
        pltpu.sync_copy(
            acc_sh.at[pl.ds(NUM_SUBCORES * chunk, tail)],
            acc_out.at[cid].at[pl.ds(NUM_SUBCORES * chunk, tail)],
        )


def _aggregate_parts(d, tab, src, dst, zeros_nd):
    return pl.kernel(
        functools.partial(_sc_aggregate, d),
        out_type=jax.ShapeDtypeStruct((NUM_CORES, N_NODES, d), jnp.float32),
        mesh=_SC_MESH,
        scratch_types=[
            pltpu.VMEM_SHARED((N_NODES, d), jnp.float32),
            pltpu.VMEM((EDGE_BATCH,), jnp.int32),
            pltpu.VMEM((EDGE_BATCH,), jnp.int32),
            pltpu.VMEM((EDGE_BATCH, d), jnp.float32),
            pltpu.SemaphoreType.DMA,
        ],
        compiler_params=pltpu.CompilerParams(use_tc_tiling_on_sc=False),
    )(tab, src, dst, zeros_nd)


def _tc_prescale(x_ref, w1_ref, degp_ref, xws_ref, dinv_ref):
    deg = 1.0 + degp_ref[0, :, 0] + degp_ref[1, :, 0]
    dinv = lax.rsqrt(deg)
    dinv_ref[:, 0] = dinv
    xws_ref[...] = (x_ref[...] @ w1_ref[...]) * dinv[:, None]


def _tc_middle(accp_ref, xws_ref, dinv_ref, b1_ref, w2_ref, out_ref):
    dinv = dinv_ref[:, 0][:, None]
    h = dinv * (accp_ref[0] + accp_ref[1] + xws_ref[...]) + b1_ref[...]
    h = jnp.maximum(h, 0.0)
    out_ref[...] = (h @ w2_ref[...]) * dinv


def _tc_final(accp_ref, hw2s_ref, dinv_ref, b2_ref, out_ref):
    dinv = dinv_ref[:, 0][:, None]
    logits = dinv * (accp_ref[0] + accp_ref[1] + hw2s_ref[...]) + b2_ref[...]
    m = jnp.max(logits, axis=1, keepdims=True)
    e = jnp.exp(logits - m)
    out_ref[...] = e / jnp.sum(e, axis=1, keepdims=True)


_ROW_BLK = 2000
_N_BLKS = N_NODES // _ROW_BLK


def _prescale(x, w1, degp):
    return pl.pallas_call(
        _tc_prescale,
        grid=(_N_BLKS,),
        in_specs=[
            pl.BlockSpec((_ROW_BLK, D_IN), lambda i: (i, 0)),
            pl.BlockSpec((D_IN, D_HID), lambda i: (0, 0)),
            pl.BlockSpec((NUM_CORES, _ROW_BLK, DEG_W), lambda i: (0, i, 0)),
        ],
        out_specs=[
            pl.BlockSpec((_ROW_BLK, D_HID), lambda i: (i, 0)),
            pl.BlockSpec((_ROW_BLK, 1), lambda i: (i, 0)),
        ],
        out_shape=[
            jax.ShapeDtypeStruct((N_NODES, D_HID), jnp.float32),
            jax.ShapeDtypeStruct((N_NODES, 1), jnp.float32),
        ],
    )(x, w1, degp)


def _middle(accp, xws, dinv, b1, w2):
    return pl.pallas_call(
        _tc_middle,
        grid=(_N_BLKS,),
        in_specs=[
            pl.BlockSpec((NUM_CORES, _ROW_BLK, D_HID), lambda i: (0, i, 0)),
            pl.BlockSpec((_ROW_BLK, D_HID), lambda i: (i, 0)),
            pl.BlockSpec((_ROW_BLK, 1), lambda i: (i, 0)),
            pl.BlockSpec((1, D_HID), lambda i: (0, 0)),
            pl.BlockSpec((D_HID, N_CLS), lambda i: (0, 0)),
        ],
        out_specs=pl.BlockSpec((_ROW_BLK, N_CLS), lambda i: (i, 0)),
        out_shape=jax.ShapeDtypeStruct((N_NODES, N_CLS), jnp.float32),
    )(accp, xws, dinv, b1, w2)


def _final(accp, hw2s, dinv, b2):
    return pl.pallas_call(
        _tc_final,
        grid=(_N_BLKS,),
        in_specs=[
            pl.BlockSpec((NUM_CORES, _ROW_BLK, N_CLS), lambda i: (0, i, 0)),
            pl.BlockSpec((_ROW_BLK, N_CLS), lambda i: (i, 0)),
            pl.BlockSpec((_ROW_BLK, 1), lambda i: (i, 0)),
            pl.BlockSpec((1, N_CLS), lambda i: (0, 0)),
        ],
        out_specs=pl.BlockSpec((_ROW_BLK, N_CLS), lambda i: (i, 0)),
        out_shape=jax.ShapeDtypeStruct((N_NODES, N_CLS), jnp.float32),
    )(accp, hw2s, dinv, b2)


@jax.jit
def kernel(node_features, edge_index, W1, b1, W2, b2):
    src = edge_index[0]
    dst = edge_index[1]
    zeros_nw = jnp.zeros((N_NODES, DEG_W), jnp.float32)
    ones_bw = jnp.ones((EDGE_BATCH, DEG_W), jnp.float32)
    zeros_nh = jnp.zeros((N_NODES, D_HID), jnp.float32)
    zeros_nc = jnp.zeros((N_NODES, N_CLS), jnp.float32)

    degp = _degree_parts(dst, zeros_nw, ones_bw)
    xws, dinv = _prescale(node_features, W1, degp)
    accp1 = _aggregate_parts(D_HID, xws, src, dst, zeros_nh)
    hw2s = _middle(accp1, xws, dinv, b1.reshape(1, D_HID), W2)
    accp2 = _aggregate_parts(N_CLS, hw2s, src, dst, zeros_nc)
    return _final(accp2, hw2s, dinv, b2.reshape(1, N_CLS))


# speedup vs baseline: 33.2504x; 2.2362x over previous
import functools
import jax
import jax.numpy as jnp
from jax import lax
from jax.experimental import pallas as pl
from jax.experimental.pallas import tpu as pltpu
from jax.experimental.pallas import tpu_sc as plsc

N_NODES = 10000
N_EDGES = 320000
D_IN = 128
D_HID = 128
N_CLS = 16

NUM_CORES = 2
NUM_SUBCORES = 16
NUM_WORKERS = NUM_CORES * NUM_SUBCORES
EDGES_PER_CORE = N_EDGES // NUM_CORES
EDGES_PER_WORKER = EDGES_PER_CORE // NUM_SUBCORES
EDGE_BATCH = 40
BATCHES_PER_WORKER = EDGES_PER_WORKER // EDGE_BATCH

_SC_MESH = plsc.VectorSubcoreMesh(
    core_axis_name="c", subcore_axis_name="s",
    num_cores=NUM_CORES, num_subcores=NUM_SUBCORES)


PIPE = 5
NUM_ROUNDS = BATCHES_PER_WORKER // PIPE
_ZCHUNK = 624
_ZTAIL = N_NODES - NUM_SUBCORES * _ZCHUNK


def _zero_init(zeros_hbm, acc_sh, sid):
    off = pl.multiple_of(sid * _ZCHUNK, 8)
    pltpu.sync_copy(zeros_hbm.at[pl.ds(off, _ZCHUNK)],
                    acc_sh.at[pl.ds(off, _ZCHUNK)])

    @pl.when(sid == NUM_SUBCORES - 1)
    def _():
        pltpu.sync_copy(zeros_hbm.at[pl.ds(NUM_SUBCORES * _ZCHUNK, _ZTAIL)],
                        acc_sh.at[pl.ds(NUM_SUBCORES * _ZCHUNK, _ZTAIL)])


def _copy_out(acc_sh, acc_out, cid, sid):
    off = pl.multiple_of(sid * _ZCHUNK, 8)
    pltpu.sync_copy(acc_sh.at[pl.ds(off, _ZCHUNK)],
                    acc_out.at[cid].at[pl.ds(off, _ZCHUNK)])

    @pl.when(sid == NUM_SUBCORES - 1)
    def _():
        pltpu.sync_copy(acc_sh.at[pl.ds(NUM_SUBCORES * _ZCHUNK, _ZTAIL)],
                        acc_out.at[cid].at[pl.ds(NUM_SUBCORES * _ZCHUNK, _ZTAIL)])


def _sc_degree(dstg_hbm, zeros_hbm, ones_hbm, deg_out, deg_sh, dst_v, ones_v,
               *sems):
    cid = lax.axis_index("c")
    sid = lax.axis_index("s")
    w = cid * NUM_SUBCORES + sid

    _zero_init(zeros_hbm, deg_sh, sid)
    pltpu.sync_copy(ones_hbm, ones_v)
    pltpu.sync_copy(dstg_hbm.at[w], dst_v)
    plsc.subcore_barrier()

    def round_(g, _):
        descs = []
        for b in range(PIPE):
            j = g * PIPE + b
            descs.append(pltpu.async_copy(
                ones_v, deg_sh.at[dst_v.at[j]], sems[b], add=True))
        for dsc in descs:
            dsc.wait()
        return 0

    lax.fori_loop(0, NUM_ROUNDS, round_, 0)
    plsc.subcore_barrier()
    _copy_out(deg_sh, deg_out, cid, sid)


DEG_W = 16


def _degree_parts(dstg, zeros_nw, ones_bw):
    return pl.kernel(
        _sc_degree,
        out_type=jax.ShapeDtypeStruct((NUM_CORES, N_NODES, DEG_W), jnp.float32),
        mesh=_SC_MESH,
        scratch_types=[
            pltpu.VMEM_SHARED((N_NODES, DEG_W), jnp.float32),
            pltpu.VMEM((BATCHES_PER_WORKER, EDGE_BATCH), jnp.int32),
            pltpu.VMEM((EDGE_BATCH, DEG_W), jnp.float32),
        ] + [pltpu.SemaphoreType.DMA] * PIPE,
        compiler_params=pltpu.CompilerParams(use_tc_tiling_on_sc=False),
    )(dstg, zeros_nw, ones_bw)


def _sc_aggregate(d, tab_hbm, srcg_hbm, dstg_hbm, zeros_hbm, acc_out,
                  acc_sh, src_v, dst_v, rows_v, *sems):
    cid = lax.axis_index("c")
    sid = lax.axis_index("s")
    w = cid * NUM_SUBCORES + sid
    gsems = sems[:PIPE]
    ssems = sems[PIPE:]

    _zero_init(zeros_hbm, acc_sh, sid)
    pltpu.sync_copy(srcg_hbm.at[w], src_v)
    pltpu.sync_copy(dstg_hbm.at[w], dst_v)
    plsc.subcore_barrier()

    def round_(g, _):
        gd = []
        for b in range(PIPE):
            j = g * PIPE + b
            gd.append(pltpu.async_copy(
                tab_hbm.at[src_v.at[j]], rows_v.at[b], gsems[b]))
        sd = []
        for b in range(PIPE):
            j = g * PIPE + b
            gd[b].wait()
            sd.append(pltpu.async_copy(
                rows_v.at[b], acc_sh.at[dst_v.at[j]], ssems[b], add=True))
        for dsc in sd:
            dsc.wait()
        return 0

    lax.fori_loop(0, NUM_ROUNDS, round_, 0)
    plsc.subcore_barrier()
    _copy_out(acc_sh, acc_out, cid, sid)


def _aggregate_parts(d, tab, srcg, dstg, zeros_nd):
    return pl.kernel(
        functools.partial(_sc_aggregate, d),
        out_type=jax.ShapeDtypeStruct((NUM_CORES, N_NODES, d), jnp.float32),
        mesh=_SC_MESH,
        scratch_types=[
            pltpu.VMEM_SHARED((N_NODES, d), jnp.float32),
            pltpu.VMEM((BATCHES_PER_WORKER, EDGE_BATCH), jnp.int32),
            pltpu.VMEM((BATCHES_PER_WORKER, EDGE_BATCH), jnp.int32),
            pltpu.VMEM((PIPE, EDGE_BATCH, d), jnp.float32),
        ] + [pltpu.SemaphoreType.DMA] * (2 * PIPE),
        compiler_params=pltpu.CompilerParams(use_tc_tiling_on_sc=False),
    )(tab, srcg, dstg, zeros_nd)


def _tc_prescale(x_ref, w1_ref, degp_ref, xws_ref, dinv_ref):
    deg = 1.0 + degp_ref[0, :, 0] + degp_ref[1, :, 0]
    dinv = lax.rsqrt(deg)
    dinv_ref[:, 0] = dinv
    xws_ref[...] = (x_ref[...] @ w1_ref[...]) * dinv[:, None]


def _tc_middle(accp_ref, xws_ref, dinv_ref, b1_ref, w2_ref, out_ref):
    dinv = dinv_ref[:, 0][:, None]
    h = dinv * (accp_ref[0] + accp_ref[1] + xws_ref[...]) + b1_ref[...]
    h = jnp.maximum(h, 0.0)
    out_ref[...] = (h @ w2_ref[...]) * dinv


def _tc_final(accp_ref, hw2s_ref, dinv_ref, b2_ref, out_ref):
    dinv = dinv_ref[:, 0][:, None]
    logits = dinv * (accp_ref[0] + accp_ref[1] + hw2s_ref[...]) + b2_ref[...]
    m = jnp.max(logits, axis=1, keepdims=True)
    e = jnp.exp(logits - m)
    out_ref[...] = e / jnp.sum(e, axis=1, keepdims=True)


_ROW_BLK = 2000
_N_BLKS = N_NODES // _ROW_BLK


def _prescale(x, w1, degp):
    return pl.pallas_call(
        _tc_prescale,
        grid=(_N_BLKS,),
        in_specs=[
            pl.BlockSpec((_ROW_BLK, D_IN), lambda i: (i, 0)),
            pl.BlockSpec((D_IN, D_HID), lambda i: (0, 0)),
            pl.BlockSpec((NUM_CORES, _ROW_BLK, DEG_W), lambda i: (0, i, 0)),
        ],
        out_specs=[
            pl.BlockSpec((_ROW_BLK, D_HID), lambda i: (i, 0)),
            pl.BlockSpec((_ROW_BLK, 1), lambda i: (i, 0)),
        ],
        out_shape=[
            jax.ShapeDtypeStruct((N_NODES, D_HID), jnp.float32),
            jax.ShapeDtypeStruct((N_NODES, 1), jnp.float32),
        ],
    )(x, w1, degp)


def _middle(accp, xws, dinv, b1, w2):
    return pl.pallas_call(
        _tc_middle,
        grid=(_N_BLKS,),
        in_specs=[
            pl.BlockSpec((NUM_CORES, _ROW_BLK, D_HID), lambda i: (0, i, 0)),
            pl.BlockSpec((_ROW_BLK, D_HID), lambda i: (i, 0)),
            pl.BlockSpec((_ROW_BLK, 1), lambda i: (i, 0)),
            pl.BlockSpec((1, D_HID), lambda i: (0, 0)),
            pl.BlockSpec((D_HID, N_CLS), lambda i: (0, 0)),
        ],
        out_specs=pl.BlockSpec((_ROW_BLK, N_CLS), lambda i: (i, 0)),
        out_shape=jax.ShapeDtypeStruct((N_NODES, N_CLS), jnp.float32),
    )(accp, xws, dinv, b1, w2)


def _final(accp, hw2s, dinv, b2):
    return pl.pallas_call(
        _tc_final,
        grid=(_N_BLKS,),
        in_specs=[
            pl.BlockSpec((NUM_CORES, _ROW_BLK, N_CLS), lambda i: (0, i, 0)),
            pl.BlockSpec((_ROW_BLK, N_CLS), lambda i: (i, 0)),
            pl.BlockSpec((_ROW_BLK, 1), lambda i: (i, 0)),
            pl.BlockSpec((1, N_CLS), lambda i: (0, 0)),
        ],
        out_specs=pl.BlockSpec((_ROW_BLK, N_CLS), lambda i: (i, 0)),
        out_shape=jax.ShapeDtypeStruct((N_NODES, N_CLS), jnp.float32),
    )(accp, hw2s, dinv, b2)


@jax.jit
def kernel(node_features, edge_index, W1, b1, W2, b2):
    grp = (NUM_WORKERS, BATCHES_PER_WORKER, EDGE_BATCH)
    srcg = edge_index[0].reshape(grp)
    dstg = edge_index[1].reshape(grp)
    zeros_nw = jnp.zeros((N_NODES, DEG_W), jnp.float32)
    ones_bw = jnp.ones((EDGE_BATCH, DEG_W), jnp.float32)
    zeros_nh = jnp.zeros((N_NODES, D_HID), jnp.float32)
    zeros_nc = jnp.zeros((N_NODES, N_CLS), jnp.float32)

    degp = _degree_parts(dstg, zeros_nw, ones_bw)
    xws, dinv = _prescale(node_features, W1, degp)
    accp1 = _aggregate_parts(D_HID, xws, srcg, dstg, zeros_nh)
    hw2s = _middle(accp1, xws, dinv, b1.reshape(1, D_HID), W2)
    accp2 = _aggregate_parts(N_CLS, hw2s, srcg, dstg, zeros_nc)
    return _final(accp2, hw2s, dinv, b2.reshape(1, N_CLS))


# speedup vs baseline: 40.4007x; 1.2150x over previous
import functools
import jax
import jax.numpy as jnp
from jax import lax
from jax.experimental import pallas as pl
from jax.experimental.pallas import tpu as pltpu
from jax.experimental.pallas import tpu_sc as plsc

N_NODES = 10000
N_EDGES = 320000
D_IN = 128
D_HID = 128
N_CLS = 16

NUM_CORES = 2
NUM_SUBCORES = 16
NUM_WORKERS = NUM_CORES * NUM_SUBCORES
EDGES_PER_CORE = N_EDGES // NUM_CORES
EDGES_PER_WORKER = EDGES_PER_CORE // NUM_SUBCORES
PIPE = 5

BATCH_W = 40
NB_W = EDGES_PER_WORKER // BATCH_W
BATCH_N = 100
NB_N = EDGES_PER_WORKER // BATCH_N

_SC_MESH = plsc.VectorSubcoreMesh(
    core_axis_name="c", subcore_axis_name="s",
    num_cores=NUM_CORES, num_subcores=NUM_SUBCORES)

_ZCHUNK = 624
_ZTAIL = N_NODES - NUM_SUBCORES * _ZCHUNK


def _zero_init(zeros_hbm, acc_sh, sid):
    off = pl.multiple_of(sid * _ZCHUNK, 8)
    pltpu.sync_copy(zeros_hbm.at[pl.ds(off, _ZCHUNK)],
                    acc_sh.at[pl.ds(off, _ZCHUNK)])

    @pl.when(sid == NUM_SUBCORES - 1)
    def _():
        pltpu.sync_copy(zeros_hbm.at[pl.ds(NUM_SUBCORES * _ZCHUNK, _ZTAIL)],
                        acc_sh.at[pl.ds(NUM_SUBCORES * _ZCHUNK, _ZTAIL)])


def _copy_out(acc_sh, acc_out, cid, sid):
    off = pl.multiple_of(sid * _ZCHUNK, 8)
    pltpu.sync_copy(acc_sh.at[pl.ds(off, _ZCHUNK)],
                    acc_out.at[cid].at[pl.ds(off, _ZCHUNK)])

    @pl.when(sid == NUM_SUBCORES - 1)
    def _():
        pltpu.sync_copy(acc_sh.at[pl.ds(NUM_SUBCORES * _ZCHUNK, _ZTAIL)],
                        acc_out.at[cid].at[pl.ds(NUM_SUBCORES * _ZCHUNK, _ZTAIL)])


DEG_W = 16


def _sc_degree(dstg_hbm, zeros_hbm, ones_hbm, deg_out, deg_sh, dst_v, ones_v,
               *sems):
    cid = lax.axis_index("c")
    sid = lax.axis_index("s")
    w = cid * NUM_SUBCORES + sid

    _zero_init(zeros_hbm, deg_sh, sid)
    pltpu.sync_copy(ones_hbm, ones_v)
    pltpu.sync_copy(dstg_hbm.at[w], dst_v)
    plsc.subcore_barrier()

    for b in range(PIPE):
        pltpu.async_copy(ones_v, deg_sh.at[dst_v.at[b]], sems[b], add=True)

    def round_(g, _):
        for b in range(PIPE):
            j = g * PIPE + b
            pltpu.make_async_copy(ones_v, deg_sh.at[dst_v.at[j]],
                                  sems[b]).wait()
            pltpu.async_copy(ones_v, deg_sh.at[dst_v.at[j]], sems[b],
                             add=True)
        return 0

    lax.fori_loop(1, NB_N // PIPE, round_, 0)
    for b in range(PIPE):
        j = NB_N - PIPE + b
        pltpu.make_async_copy(ones_v, deg_sh.at[dst_v.at[j]], sems[b]).wait()
    plsc.subcore_barrier()
    _copy_out(deg_sh, deg_out, cid, sid)


def _degree_parts(dstg, zeros_nw, ones_bw):
    return pl.kernel(
        _sc_degree,
        out_type=jax.ShapeDtypeStruct((NUM_CORES, N_NODES, DEG_W), jnp.float32),
        mesh=_SC_MESH,
        scratch_types=[
            pltpu.VMEM_SHARED((N_NODES, DEG_W), jnp.float32),
            pltpu.VMEM((NB_N, BATCH_N), jnp.int32),
            pltpu.VMEM((BATCH_N, DEG_W), jnp.float32),
        ] + [pltpu.SemaphoreType.DMA] * PIPE,
        compiler_params=pltpu.CompilerParams(use_tc_tiling_on_sc=False),
    )(dstg, zeros_nw, ones_bw)


def _sc_aggregate(d, batch, nb, tab_hbm, srcg_hbm, dstg_hbm, zeros_hbm,
                  acc_out, acc_sh, src_v, dst_v, rows_v, *sems):
    cid = lax.axis_index("c")
    sid = lax.axis_index("s")
    w = cid * NUM_SUBCORES + sid
    gsems = sems[:PIPE]
    ssems = sems[PIPE:]

    _zero_init(zeros_hbm, acc_sh, sid)
    pltpu.sync_copy(srcg_hbm.at[w], src_v)
    pltpu.sync_copy(dstg_hbm.at[w], dst_v)
    plsc.subcore_barrier()

    for b in range(PIPE):
        pltpu.async_copy(tab_hbm.at[src_v.at[b]], rows_v.at[b], gsems[b])

    rounds = nb // PIPE

    def round_(g, _):
        sd = []
        for b in range(PIPE):
            j = g * PIPE + b
            pltpu.make_async_copy(tab_hbm.at[src_v.at[j]], rows_v.at[b],
                                  gsems[b]).wait()
            sd.append(pltpu.async_copy(
                rows_v.at[b], acc_sh.at[dst_v.at[j]], ssems[b], add=True))
        for b in range(PIPE):
            sd[b].wait()

            @pl.when(g < rounds - 1)
            def _(b=b):
                jn = (g + 1) * PIPE + b
                pltpu.async_copy(tab_hbm.at[src_v.at[jn]], rows_v.at[b],
                                 gsems[b])
        return 0

    lax.fori_loop(0, rounds, round_, 0)
    plsc.subcore_barrier()
    _copy_out(acc_sh, acc_out, cid, sid)


def _aggregate_parts(d, batch, nb, tab, srcg, dstg, zeros_nd):
    return pl.kernel(
        functools.partial(_sc_aggregate, d, batch, nb),
        out_type=jax.ShapeDtypeStruct((NUM_CORES, N_NODES, d), jnp.float32),
        mesh=_SC_MESH,
        scratch_types=[
            pltpu.VMEM_SHARED((N_NODES, d), jnp.float32),
            pltpu.VMEM((nb, batch), jnp.int32),
            pltpu.VMEM((nb, batch), jnp.int32),
            pltpu.VMEM((PIPE, batch, d), jnp.float32),
        ] + [pltpu.SemaphoreType.DMA] * (2 * PIPE),
        compiler_params=pltpu.CompilerParams(use_tc_tiling_on_sc=False),
    )(tab, srcg, dstg, zeros_nd)


def _tc_prescale(x_ref, w1_ref, degp_ref, xws_ref, dinv_ref):
    deg = 1.0 + degp_ref[0, :, 0] + degp_ref[1, :, 0]
    dinv = lax.rsqrt(deg)
    dinv_ref[:, 0] = dinv
    xws_ref[...] = (x_ref[...] @ w1_ref[...]) * dinv[:, None]


def _tc_middle(accp_ref, xws_ref, dinv_ref, b1_ref, w2_ref, out_ref):
    dinv = dinv_ref[:, 0][:, None]
    h = dinv * (accp_ref[0] + accp_ref[1] + xws_ref[...]) + b1_ref[...]
    h = jnp.maximum(h, 0.0)
    out_ref[...] = (h @ w2_ref[...]) * dinv


def _tc_final(accp_ref, hw2s_ref, dinv_ref, b2_ref, out_ref):
    dinv = dinv_ref[:, 0][:, None]
    logits = dinv * (accp_ref[0] + accp_ref[1] + hw2s_ref[...]) + b2_ref[...]
    m = jnp.max(logits, axis=1, keepdims=True)
    e = jnp.exp(logits - m)
    out_ref[...] = e / jnp.sum(e, axis=1, keepdims=True)


_ROW_BLK = 2000
_N_BLKS = N_NODES // _ROW_BLK


def _prescale(x, w1, degp):
    return pl.pallas_call(
        _tc_prescale,
        grid=(_N_BLKS,),
        in_specs=[
            pl.BlockSpec((_ROW_BLK, D_IN), lambda i: (i, 0)),
            pl.BlockSpec((D_IN, D_HID), lambda i: (0, 0)),
            pl.BlockSpec((NUM_CORES, _ROW_BLK, DEG_W), lambda i: (0, i, 0)),
        ],
        out_specs=[
            pl.BlockSpec((_ROW_BLK, D_HID), lambda i: (i, 0)),
            pl.BlockSpec((_ROW_BLK, 1), lambda i: (i, 0)),
        ],
        out_shape=[
            jax.ShapeDtypeStruct((N_NODES, D_HID), jnp.float32),
            jax.ShapeDtypeStruct((N_NODES, 1), jnp.float32),
        ],
    )(x, w1, degp)


def _middle(accp, xws, dinv, b1, w2):
    return pl.pallas_call(
        _tc_middle,
        grid=(_N_BLKS,),
        in_specs=[
            pl.BlockSpec((NUM_CORES, _ROW_BLK, D_HID), lambda i: (0, i, 0)),
            pl.BlockSpec((_ROW_BLK, D_HID), lambda i: (i, 0)),
            pl.BlockSpec((_ROW_BLK, 1), lambda i: (i, 0)),
            pl.BlockSpec((1, D_HID), lambda i: (0, 0)),
            pl.BlockSpec((D_HID, N_CLS), lambda i: (0, 0)),
        ],
        out_specs=pl.BlockSpec((_ROW_BLK, N_CLS), lambda i: (i, 0)),
        out_shape=jax.ShapeDtypeStruct((N_NODES, N_CLS), jnp.float32),
    )(accp, xws, dinv, b1, w2)


def _final(accp, hw2s, dinv, b2):
    return pl.pallas_call(
        _tc_final,
        grid=(_N_BLKS,),
        in_specs=[
            pl.BlockSpec((NUM_CORES, _ROW_BLK, N_CLS), lambda i: (0, i, 0)),
            pl.BlockSpec((_ROW_BLK, N_CLS), lambda i: (i, 0)),
            pl.BlockSpec((_ROW_BLK, 1), lambda i: (i, 0)),
            pl.BlockSpec((1, N_CLS), lambda i: (0, 0)),
        ],
        out_specs=pl.BlockSpec((_ROW_BLK, N_CLS), lambda i: (i, 0)),
        out_shape=jax.ShapeDtypeStruct((N_NODES, N_CLS), jnp.float32),
    )(accp, hw2s, dinv, b2)


@jax.jit
def kernel(node_features, edge_index, W1, b1, W2, b2):
    grp_w = (NUM_WORKERS, NB_W, BATCH_W)
    grp_n = (NUM_WORKERS, NB_N, BATCH_N)
    srcg_w = edge_index[0].reshape(grp_w)
    dstg_w = edge_index[1].reshape(grp_w)
    srcg_n = edge_index[0].reshape(grp_n)
    dstg_n = edge_index[1].reshape(grp_n)
    zeros_nw = jnp.zeros((N_NODES, DEG_W), jnp.float32)
    ones_bw = jnp.ones((BATCH_N, DEG_W), jnp.float32)
    zeros_nh = jnp.zeros((N_NODES, D_HID), jnp.float32)
    zeros_nc = jnp.zeros((N_NODES, N_CLS), jnp.float32)

    degp = _degree_parts(dstg_n, zeros_nw, ones_bw)
    xws, dinv = _prescale(node_features, W1, degp)
    accp1 = _aggregate_parts(D_HID, BATCH_W, NB_W, xws, srcg_w, dstg_w,
                             zeros_nh)
    hw2s = _middle(accp1, xws, dinv, b1.reshape(1, D_HID), W2)
    accp2 = _aggregate_parts(N_CLS, BATCH_N, NB_N, hw2s, srcg_n, dstg_n,
                             zeros_nc)
    return _final(accp2, hw2s, dinv, b2.reshape(1, N_CLS))


# speedup vs baseline: 40.6809x; 1.0069x over previous
import functools
import jax
import jax.numpy as jnp
from jax import lax
from jax.experimental import pallas as pl
from jax.experimental.pallas import tpu as pltpu
from jax.experimental.pallas import tpu_sc as plsc

N_NODES = 10000
N_EDGES = 320000
D_IN = 128
D_HID = 128
N_CLS = 16

NUM_CORES = 2
NUM_SUBCORES = 16
NUM_WORKERS = NUM_CORES * NUM_SUBCORES
EDGES_PER_CORE = N_EDGES // NUM_CORES
EDGES_PER_WORKER = EDGES_PER_CORE // NUM_SUBCORES
PIPE_W = 5
PIPE_N = 10

BATCH_W = 40
NB_W = EDGES_PER_WORKER // BATCH_W
BATCH_N = 100
NB_N = EDGES_PER_WORKER // BATCH_N

_SC_MESH = plsc.VectorSubcoreMesh(
    core_axis_name="c", subcore_axis_name="s",
    num_cores=NUM_CORES, num_subcores=NUM_SUBCORES)

_ZCHUNK = 624
_ZTAIL = N_NODES - NUM_SUBCORES * _ZCHUNK


def _zero_init(zeros_hbm, acc_sh, sid):
    off = pl.multiple_of(sid * _ZCHUNK, 8)
    pltpu.sync_copy(zeros_hbm.at[pl.ds(off, _ZCHUNK)],
                    acc_sh.at[pl.ds(off, _ZCHUNK)])

    @pl.when(sid == NUM_SUBCORES - 1)
    def _():
        pltpu.sync_copy(zeros_hbm.at[pl.ds(NUM_SUBCORES * _ZCHUNK, _ZTAIL)],
                        acc_sh.at[pl.ds(NUM_SUBCORES * _ZCHUNK, _ZTAIL)])


def _copy_out(acc_sh, acc_out, cid, sid):
    off = pl.multiple_of(sid * _ZCHUNK, 8)
    pltpu.sync_copy(acc_sh.at[pl.ds(off, _ZCHUNK)],
                    acc_out.at[cid].at[pl.ds(off, _ZCHUNK)])

    @pl.when(sid == NUM_SUBCORES - 1)
    def _():
        pltpu.sync_copy(acc_sh.at[pl.ds(NUM_SUBCORES * _ZCHUNK, _ZTAIL)],
                        acc_out.at[cid].at[pl.ds(NUM_SUBCORES * _ZCHUNK, _ZTAIL)])


DEG_W = 16


def _sc_degree(dstg_hbm, zeros_hbm, ones_hbm, deg_out, deg_sh, dst_v, ones_v,
               *sems):
    cid = lax.axis_index("c")
    sid = lax.axis_index("s")
    w = cid * NUM_SUBCORES + sid

    _zero_init(zeros_hbm, deg_sh, sid)
    pltpu.sync_copy(ones_hbm, ones_v)
    pltpu.sync_copy(dstg_hbm.at[w], dst_v)
    plsc.subcore_barrier()

    for b in range(PIPE_N):
        pltpu.async_copy(ones_v, deg_sh.at[dst_v.at[b]], sems[b], add=True)

    def round_(g, _):
        for b in range(PIPE_N):
            j = g * PIPE_N + b
            pltpu.make_async_copy(ones_v, deg_sh.at[dst_v.at[j]],
                                  sems[b]).wait()
            pltpu.async_copy(ones_v, deg_sh.at[dst_v.at[j]], sems[b],
                             add=True)
        return 0

    lax.fori_loop(1, NB_N // PIPE_N, round_, 0)
    for b in range(PIPE_N):
        j = NB_N - PIPE_N + b
        pltpu.make_async_copy(ones_v, deg_sh.at[dst_v.at[j]], sems[b]).wait()
    plsc.subcore_barrier()
    _copy_out(deg_sh, deg_out, cid, sid)


def _degree_parts(dstg, zeros_nw, ones_bw):
    return pl.kernel(
        _sc_degree,
        out_type=jax.ShapeDtypeStruct((NUM_CORES, N_NODES, DEG_W), jnp.float32),
        mesh=_SC_MESH,
        scratch_types=[
            pltpu.VMEM_SHARED((N_NODES, DEG_W), jnp.float32),
            pltpu.VMEM((NB_N, BATCH_N), jnp.int32),
            pltpu.VMEM((BATCH_N, DEG_W), jnp.float32),
        ] + [pltpu.SemaphoreType.DMA] * PIPE_N,
        compiler_params=pltpu.CompilerParams(use_tc_tiling_on_sc=False),
    )(dstg, zeros_nw, ones_bw)


def _sc_aggregate(d, batch, nb, pipe, tab_hbm, srcg_hbm, dstg_hbm, zeros_hbm,
                  acc_out, acc_sh, src_v, dst_v, rows_v, *sems):
    cid = lax.axis_index("c")
    sid = lax.axis_index("s")
    w = cid * NUM_SUBCORES + sid
    gsems = sems[:pipe]
    ssems = sems[pipe:]

    _zero_init(zeros_hbm, acc_sh, sid)
    pltpu.sync_copy(srcg_hbm.at[w], src_v)
    pltpu.sync_copy(dstg_hbm.at[w], dst_v)
    plsc.subcore_barrier()

    for b in range(pipe):
        pltpu.async_copy(tab_hbm.at[src_v.at[b]], rows_v.at[b], gsems[b])

    rounds = nb // pipe

    def round_(g, _):
        sd = []
        for b in range(pipe):
            j = g * pipe + b
            pltpu.make_async_copy(tab_hbm.at[src_v.at[j]], rows_v.at[b],
                                  gsems[b]).wait()
            sd.append(pltpu.async_copy(
                rows_v.at[b], acc_sh.at[dst_v.at[j]], ssems[b], add=True))
        for b in range(pipe):
            sd[b].wait()

            @pl.when(g < rounds - 1)
            def _(b=b):
                jn = (g + 1) * pipe + b
                pltpu.async_copy(tab_hbm.at[src_v.at[jn]], rows_v.at[b],
                                 gsems[b])
        return 0

    lax.fori_loop(0, rounds, round_, 0)
    plsc.subcore_barrier()
    _copy_out(acc_sh, acc_out, cid, sid)


def _aggregate_parts(d, batch, nb, pipe, tab, srcg, dstg, zeros_nd):
    return pl.kernel(
        functools.partial(_sc_aggregate, d, batch, nb, pipe),
        out_type=jax.ShapeDtypeStruct((NUM_CORES, N_NODES, d), jnp.float32),
        mesh=_SC_MESH,
        scratch_types=[
            pltpu.VMEM_SHARED((N_NODES, d), jnp.float32),
            pltpu.VMEM((nb, batch), jnp.int32),
            pltpu.VMEM((nb, batch), jnp.int32),
            pltpu.VMEM((pipe, batch, d), jnp.float32),
        ] + [pltpu.SemaphoreType.DMA] * (2 * pipe),
        compiler_params=pltpu.CompilerParams(use_tc_tiling_on_sc=False),
    )(tab, srcg, dstg, zeros_nd)


def _tc_matmul1(x_ref, w1_ref, xw_ref):
    xw_ref[...] = x_ref[...] @ w1_ref[...]


def _tc_scale(xw_ref, degp_ref, xws_ref, dinv_ref):
    deg = 1.0 + degp_ref[0, :, 0] + degp_ref[1, :, 0]
    dinv = lax.rsqrt(deg)
    dinv_ref[:, 0] = dinv
    xws_ref[...] = xw_ref[...] * dinv[:, None]


def _tc_middle(accp_ref, xws_ref, dinv_ref, b1_ref, w2_ref, out_ref):
    dinv = dinv_ref[:, 0][:, None]
    h = dinv * (accp_ref[0] + accp_ref[1] + xws_ref[...]) + b1_ref[...]
    h = jnp.maximum(h, 0.0)
    out_ref[...] = (h @ w2_ref[...]) * dinv


def _tc_final(accp_ref, hw2s_ref, dinv_ref, b2_ref, out_ref):
    dinv = dinv_ref[:, 0][:, None]
    logits = dinv * (accp_ref[0] + accp_ref[1] + hw2s_ref[...]) + b2_ref[...]
    m = jnp.max(logits, axis=1, keepdims=True)
    e = jnp.exp(logits - m)
    out_ref[...] = e / jnp.sum(e, axis=1, keepdims=True)


_ROW_BLK = 2000
_N_BLKS = N_NODES // _ROW_BLK


def _matmul1(x, w1):
    return pl.pallas_call(
        _tc_matmul1,
        grid=(_N_BLKS,),
        in_specs=[
            pl.BlockSpec((_ROW_BLK, D_IN), lambda i: (i, 0)),
            pl.BlockSpec((D_IN, D_HID), lambda i: (0, 0)),
        ],
        out_specs=pl.BlockSpec((_ROW_BLK, D_HID), lambda i: (i, 0)),
        out_shape=jax.ShapeDtypeStruct((N_NODES, D_HID), jnp.float32),
    )(x, w1)


def _scale(xw, degp):
    return pl.pallas_call(
        _tc_scale,
        grid=(_N_BLKS,),
        in_specs=[
            pl.BlockSpec((_ROW_BLK, D_HID), lambda i: (i, 0)),
            pl.BlockSpec((NUM_CORES, _ROW_BLK, DEG_W), lambda i: (0, i, 0)),
        ],
        out_specs=[
            pl.BlockSpec((_ROW_BLK, D_HID), lambda i: (i, 0)),
            pl.BlockSpec((_ROW_BLK, 1), lambda i: (i, 0)),
        ],
        out_shape=[
            jax.ShapeDtypeStruct((N_NODES, D_HID), jnp.float32),
            jax.ShapeDtypeStruct((N_NODES, 1), jnp.float32),
        ],
    )(xw, degp)


def _middle(accp, xws, dinv, b1, w2):
    return pl.pallas_call(
        _tc_middle,
        grid=(_N_BLKS,),
        in_specs=[
            pl.BlockSpec((NUM_CORES, _ROW_BLK, D_HID), lambda i: (0, i, 0)),
            pl.BlockSpec((_ROW_BLK, D_HID), lambda i: (i, 0)),
            pl.BlockSpec((_ROW_BLK, 1), lambda i: (i, 0)),
            pl.BlockSpec((1, D_HID), lambda i: (0, 0)),
            pl.BlockSpec((D_HID, N_CLS), lambda i: (0, 0)),
        ],
        out_specs=pl.BlockSpec((_ROW_BLK, N_CLS), lambda i: (i, 0)),
        out_shape=jax.ShapeDtypeStruct((N_NODES, N_CLS), jnp.float32),
    )(accp, xws, dinv, b1, w2)


def _final(accp, hw2s, dinv, b2):
    return pl.pallas_call(
        _tc_final,
        grid=(_N_BLKS,),
        in_specs=[
            pl.BlockSpec((NUM_CORES, _ROW_BLK, N_CLS), lambda i: (0, i, 0)),
            pl.BlockSpec((_ROW_BLK, N_CLS), lambda i: (i, 0)),
            pl.BlockSpec((_ROW_BLK, 1), lambda i: (i, 0)),
            pl.BlockSpec((1, N_CLS), lambda i: (0, 0)),
        ],
        out_specs=pl.BlockSpec((_ROW_BLK, N_CLS), lambda i: (i, 0)),
        out_shape=jax.ShapeDtypeStruct((N_NODES, N_CLS), jnp.float32),
    )(accp, hw2s, dinv, b2)


@jax.jit
def kernel(node_features, edge_index, W1, b1, W2, b2):
    grp_w = (NUM_WORKERS, NB_W, BATCH_W)
    grp_n = (NUM_WORKERS, NB_N, BATCH_N)
    srcg_w = edge_index[0].reshape(grp_w)
    dstg_w = edge_index[1].reshape(grp_w)
    srcg_n = edge_index[0].reshape(grp_n)
    dstg_n = edge_index[1].reshape(grp_n)
    zeros_nw = jnp.zeros((N_NODES, DEG_W), jnp.float32)
    ones_bw = jnp.ones((BATCH_N, DEG_W), jnp.float32)
    zeros_nh = jnp.zeros((N_NODES, D_HID), jnp.float32)
    zeros_nc = jnp.zeros((N_NODES, N_CLS), jnp.float32)

    degp = _degree_parts(dstg_n, zeros_nw, ones_bw)
    xw = _matmul1(node_features, W1)
    xws, dinv = _scale(xw, degp)
    accp1 = _aggregate_parts(D_HID, BATCH_W, NB_W, PIPE_W, xws, srcg_w,
                             dstg_w, zeros_nh)
    hw2s = _middle(accp1, xws, dinv, b1.reshape(1, D_HID), W2)
    accp2 = _aggregate_parts(N_CLS, BATCH_N, NB_N, PIPE_N, hw2s, srcg_n,
                             dstg_n, zeros_nc)
    return _final(accp2, hw2s, dinv, b2.reshape(1, N_CLS))


# speedup vs baseline: 41.5746x; 1.0220x over previous
import functools
import jax
import jax.numpy as jnp
from jax import lax
from jax.experimental import pallas as pl
from jax.experimental.pallas import tpu as pltpu
from jax.experimental.pallas import tpu_sc as plsc

N_NODES = 10000
N_EDGES = 320000
D_IN = 128
D_HID = 128
N_CLS = 16

NUM_CORES = 2
NUM_SUBCORES = 16
NUM_WORKERS = NUM_CORES * NUM_SUBCORES
EDGES_PER_CORE = N_EDGES // NUM_CORES
EDGES_PER_WORKER = EDGES_PER_CORE // NUM_SUBCORES
PIPE_W = 5
PIPE_N = 10

BATCH_W = 40
NB_W = EDGES_PER_WORKER // BATCH_W
BATCH_N = 100
NB_N = EDGES_PER_WORKER // BATCH_N

_SC_MESH = plsc.VectorSubcoreMesh(
    core_axis_name="c", subcore_axis_name="s",
    num_cores=NUM_CORES, num_subcores=NUM_SUBCORES)

_ZCHUNK = 624
_ZTAIL = N_NODES - NUM_SUBCORES * _ZCHUNK


def _zero_init(zeros_hbm, acc_sh, sid):
    off = pl.multiple_of(sid * _ZCHUNK, 8)
    pltpu.sync_copy(zeros_hbm.at[pl.ds(off, _ZCHUNK)],
                    acc_sh.at[pl.ds(off, _ZCHUNK)])

    @pl.when(sid == NUM_SUBCORES - 1)
    def _():
        pltpu.sync_copy(zeros_hbm.at[pl.ds(NUM_SUBCORES * _ZCHUNK, _ZTAIL)],
                        acc_sh.at[pl.ds(NUM_SUBCORES * _ZCHUNK, _ZTAIL)])


def _copy_out(acc_sh, acc_out, cid, sid):
    off = pl.multiple_of(sid * _ZCHUNK, 8)
    pltpu.sync_copy(acc_sh.at[pl.ds(off, _ZCHUNK)],
                    acc_out.at[cid].at[pl.ds(off, _ZCHUNK)])

    @pl.when(sid == NUM_SUBCORES - 1)
    def _():
        pltpu.sync_copy(acc_sh.at[pl.ds(NUM_SUBCORES * _ZCHUNK, _ZTAIL)],
                        acc_out.at[cid].at[pl.ds(NUM_SUBCORES * _ZCHUNK, _ZTAIL)])


DEG_W = 16


def _sc_degree(dstg_hbm, zeros_hbm, ones_hbm, deg_out, deg_sh, dst_v, ones_v,
               *sems):
    cid = lax.axis_index("c")
    sid = lax.axis_index("s")
    w = cid * NUM_SUBCORES + sid

    _zero_init(zeros_hbm, deg_sh, sid)
    pltpu.sync_copy(ones_hbm, ones_v)
    pltpu.sync_copy(dstg_hbm.at[w], dst_v)
    plsc.subcore_barrier()

    for b in range(PIPE_N):
        pltpu.async_copy(ones_v, deg_sh.at[dst_v.at[b]], sems[b], add=True)

    def round_(g, _):
        for b in range(PIPE_N):
            j = g * PIPE_N + b
            pltpu.make_async_copy(ones_v, deg_sh.at[dst_v.at[j]],
                                  sems[b]).wait()
            pltpu.async_copy(ones_v, deg_sh.at[dst_v.at[j]], sems[b],
                             add=True)
        return 0

    lax.fori_loop(1, NB_N // PIPE_N, round_, 0)
    for b in range(PIPE_N):
        j = NB_N - PIPE_N + b
        pltpu.make_async_copy(ones_v, deg_sh.at[dst_v.at[j]], sems[b]).wait()
    plsc.subcore_barrier()
    _copy_out(deg_sh, deg_out, cid, sid)


def _degree_parts(dstg, zeros_nw, ones_bw):
    return pl.kernel(
        _sc_degree,
        out_type=jax.ShapeDtypeStruct((NUM_CORES, N_NODES, DEG_W), jnp.float32),
        mesh=_SC_MESH,
        scratch_types=[
            pltpu.VMEM_SHARED((N_NODES, DEG_W), jnp.float32),
            pltpu.VMEM((NB_N, BATCH_N), jnp.int32),
            pltpu.VMEM((BATCH_N, DEG_W), jnp.float32),
        ] + [pltpu.SemaphoreType.DMA] * PIPE_N,
        compiler_params=pltpu.CompilerParams(use_tc_tiling_on_sc=False),
    )(dstg, zeros_nw, ones_bw)


def _sc_aggregate(d, batch, nb, pipe, tab_hbm, srcg_hbm, dstg_hbm, zeros_hbm,
                  acc_out, acc_sh, src_v, dst_v, rows_v, *sems):
    cid = lax.axis_index("c")
    sid = lax.axis_index("s")
    w = cid * NUM_SUBCORES + sid
    gsems = sems[:pipe]
    ssems = sems[pipe:2 * pipe]
    zc, zt, si, di = sems[2 * pipe:]

    off = pl.multiple_of(sid * _ZCHUNK, 8)
    zd = pltpu.async_copy(zeros_hbm.at[pl.ds(off, _ZCHUNK)],
                          acc_sh.at[pl.ds(off, _ZCHUNK)], zc)

    @pl.when(sid == NUM_SUBCORES - 1)
    def _():
        pltpu.async_copy(zeros_hbm.at[pl.ds(NUM_SUBCORES * _ZCHUNK, _ZTAIL)],
                         acc_sh.at[pl.ds(NUM_SUBCORES * _ZCHUNK, _ZTAIL)], zt)

    sd_src = pltpu.async_copy(srcg_hbm.at[w], src_v, si)
    sd_dst = pltpu.async_copy(dstg_hbm.at[w], dst_v, di)
    sd_src.wait()

    for b in range(pipe):
        pltpu.async_copy(tab_hbm.at[src_v.at[b]], rows_v.at[b], gsems[b])

    zd.wait()

    @pl.when(sid == NUM_SUBCORES - 1)
    def _():
        pltpu.make_async_copy(
            zeros_hbm.at[pl.ds(NUM_SUBCORES * _ZCHUNK, _ZTAIL)],
            acc_sh.at[pl.ds(NUM_SUBCORES * _ZCHUNK, _ZTAIL)], zt).wait()

    sd_dst.wait()
    plsc.subcore_barrier()

    rounds = nb // pipe

    def round_(g, _):
        sd = []
        for b in range(pipe):
            j = g * pipe + b
            pltpu.make_async_copy(tab_hbm.at[src_v.at[j]], rows_v.at[b],
                                  gsems[b]).wait()
            sd.append(pltpu.async_copy(
                rows_v.at[b], acc_sh.at[dst_v.at[j]], ssems[b], add=True))
        for b in range(pipe):
            sd[b].wait()

            @pl.when(g < rounds - 1)
            def _(b=b):
                jn = (g + 1) * pipe + b
                pltpu.async_copy(tab_hbm.at[src_v.at[jn]], rows_v.at[b],
                                 gsems[b])
        return 0

    lax.fori_loop(0, rounds, round_, 0)
    plsc.subcore_barrier()
    _copy_out(acc_sh, acc_out, cid, sid)


def _aggregate_parts(d, batch, nb, pipe, tab, srcg, dstg, zeros_nd):
    return pl.kernel(
        functools.partial(_sc_aggregate, d, batch, nb, pipe),
        out_type=jax.ShapeDtypeStruct((NUM_CORES, N_NODES, d), jnp.float32),
        mesh=_SC_MESH,
        scratch_types=[
            pltpu.VMEM_SHARED((N_NODES, d), jnp.float32),
            pltpu.VMEM((nb, batch), jnp.int32),
            pltpu.VMEM((nb, batch), jnp.int32),
            pltpu.VMEM((pipe, batch, d), jnp.float32),
        ] + [pltpu.SemaphoreType.DMA] * (2 * pipe + 4),
        compiler_params=pltpu.CompilerParams(use_tc_tiling_on_sc=False),
    )(tab, srcg, dstg, zeros_nd)


def _tc_matmul1(x_ref, w1_ref, xw_ref):
    xw_ref[...] = x_ref[...] @ w1_ref[...]


def _tc_scale(xw_ref, degp_ref, xws_ref, dinv_ref):
    deg = 1.0 + degp_ref[0, :, 0] + degp_ref[1, :, 0]
    dinv = lax.rsqrt(deg)
    dinv_ref[:, 0] = dinv
    xws_ref[...] = xw_ref[...] * dinv[:, None]


def _tc_middle(accp_ref, xws_ref, dinv_ref, b1_ref, w2_ref, out_ref):
    dinv = dinv_ref[:, 0][:, None]
    h = dinv * (accp_ref[0] + accp_ref[1] + xws_ref[...]) + b1_ref[...]
    h = jnp.maximum(h, 0.0)
    out_ref[...] = (h @ w2_ref[...]) * dinv


def _tc_final(accp_ref, hw2s_ref, dinv_ref, b2_ref, out_ref):
    dinv = dinv_ref[:, 0][:, None]
    logits = dinv * (accp_ref[0] + accp_ref[1] + hw2s_ref[...]) + b2_ref[...]
    m = jnp.max(logits, axis=1, keepdims=True)
    e = jnp.exp(logits - m)
    out_ref[...] = e / jnp.sum(e, axis=1, keepdims=True)


_ROW_BLK = 2000
_N_BLKS = N_NODES // _ROW_BLK


def _matmul1(x, w1):
    return pl.pallas_call(
        _tc_matmul1,
        grid=(_N_BLKS,),
        in_specs=[
            pl.BlockSpec((_ROW_BLK, D_IN), lambda i: (i, 0)),
            pl.BlockSpec((D_IN, D_HID), lambda i: (0, 0)),
        ],
        out_specs=pl.BlockSpec((_ROW_BLK, D_HID), lambda i: (i, 0)),
        out_shape=jax.ShapeDtypeStruct((N_NODES, D_HID), jnp.float32),
    )(x, w1)


def _scale(xw, degp):
    return pl.pallas_call(
        _tc_scale,
        grid=(_N_BLKS,),
        in_specs=[
            pl.BlockSpec((_ROW_BLK, D_HID), lambda i: (i, 0)),
            pl.BlockSpec((NUM_CORES, _ROW_BLK, DEG_W), lambda i: (0, i, 0)),
        ],
        out_specs=[
            pl.BlockSpec((_ROW_BLK, D_HID), lambda i: (i, 0)),
            pl.BlockSpec((_ROW_BLK, 1), lambda i: (i, 0)),
        ],
        out_shape=[
            jax.ShapeDtypeStruct((N_NODES, D_HID), jnp.float32),
            jax.ShapeDtypeStruct((N_NODES, 1), jnp.float32),
        ],
    )(xw, degp)


def _middle(accp, xws, dinv, b1, w2):
    return pl.pallas_call(
        _tc_middle,
        grid=(_N_BLKS,),
        in_specs=[
            pl.BlockSpec((NUM_CORES, _ROW_BLK, D_HID), lambda i: (0, i, 0)),
            pl.BlockSpec((_ROW_BLK, D_HID), lambda i: (i, 0)),
            pl.BlockSpec((_ROW_BLK, 1), lambda i: (i, 0)),
            pl.BlockSpec((1, D_HID), lambda i: (0, 0)),
            pl.BlockSpec((D_HID, N_CLS), lambda i: (0, 0)),
        ],
        out_specs=pl.BlockSpec((_ROW_BLK, N_CLS), lambda i: (i, 0)),
        out_shape=jax.ShapeDtypeStruct((N_NODES, N_CLS), jnp.float32),
    )(accp, xws, dinv, b1, w2)


def _final(accp, hw2s, dinv, b2):
    return pl.pallas_call(
        _tc_final,
        grid=(_N_BLKS,),
        in_specs=[
            pl.BlockSpec((NUM_CORES, _ROW_BLK, N_CLS), lambda i: (0, i, 0)),
            pl.BlockSpec((_ROW_BLK, N_CLS), lambda i: (i, 0)),
            pl.BlockSpec((_ROW_BLK, 1), lambda i: (i, 0)),
            pl.BlockSpec((1, N_CLS), lambda i: (0, 0)),
        ],
        out_specs=pl.BlockSpec((_ROW_BLK, N_CLS), lambda i: (i, 0)),
        out_shape=jax.ShapeDtypeStruct((N_NODES, N_CLS), jnp.float32),
    )(accp, hw2s, dinv, b2)


@jax.jit
def kernel(node_features, edge_index, W1, b1, W2, b2):
    grp_w = (NUM_WORKERS, NB_W, BATCH_W)
    grp_n = (NUM_WORKERS, NB_N, BATCH_N)
    srcg_w = edge_index[0].reshape(grp_w)
    dstg_w = edge_index[1].reshape(grp_w)
    srcg_n = edge_index[0].reshape(grp_n)
    dstg_n = edge_index[1].reshape(grp_n)
    zeros_nw = jnp.zeros((N_NODES, DEG_W), jnp.float32)
    ones_bw = jnp.ones((BATCH_N, DEG_W), jnp.float32)
    zeros_nh = jnp.zeros((N_NODES, D_HID), jnp.float32)
    zeros_nc = jnp.zeros((N_NODES, N_CLS), jnp.float32)

    degp = _degree_parts(dstg_n, zeros_nw, ones_bw)
    xw = _matmul1(node_features, W1)
    xws, dinv = _scale(xw, degp)
    accp1 = _aggregate_parts(D_HID, BATCH_W, NB_W, PIPE_W, xws, srcg_w,
                             dstg_w, zeros_nh)
    hw2s = _middle(accp1, xws, dinv, b1.reshape(1, D_HID), W2)
    accp2 = _aggregate_parts(N_CLS, BATCH_N, NB_N, PIPE_N, hw2s, srcg_n,
                             dstg_n, zeros_nc)
    return _final(accp2, hw2s, dinv, b2.reshape(1, N_CLS))


# speedup vs baseline: 41.9549x; 1.0091x over previous
import functools
import jax
import jax.numpy as jnp
from jax import lax
from jax.experimental import pallas as pl
from jax.experimental.pallas import tpu as pltpu
from jax.experimental.pallas import tpu_sc as plsc

N_NODES = 10000
N_EDGES = 320000
D_IN = 128
D_HID = 128
N_CLS = 16

NUM_CORES = 2
NUM_SUBCORES = 16
NUM_WORKERS = NUM_CORES * NUM_SUBCORES
EDGES_PER_CORE = N_EDGES // NUM_CORES
EDGES_PER_WORKER = EDGES_PER_CORE // NUM_SUBCORES
PIPE_W = 5
PIPE_N = 10

BATCH_W = 40
NB_W = EDGES_PER_WORKER // BATCH_W
BATCH_N = 100
NB_N = EDGES_PER_WORKER // BATCH_N

_SC_MESH = plsc.VectorSubcoreMesh(
    core_axis_name="c", subcore_axis_name="s",
    num_cores=NUM_CORES, num_subcores=NUM_SUBCORES)

_ZCHUNK = 624
_ZTAIL = N_NODES - NUM_SUBCORES * _ZCHUNK


def _zero_init(zeros_hbm, acc_sh, sid):
    off = pl.multiple_of(sid * _ZCHUNK, 8)
    pltpu.sync_copy(zeros_hbm.at[pl.ds(off, _ZCHUNK)],
                    acc_sh.at[pl.ds(off, _ZCHUNK)])

    @pl.when(sid == NUM_SUBCORES - 1)
    def _():
        pltpu.sync_copy(zeros_hbm.at[pl.ds(NUM_SUBCORES * _ZCHUNK, _ZTAIL)],
                        acc_sh.at[pl.ds(NUM_SUBCORES * _ZCHUNK, _ZTAIL)])


def _copy_out(acc_sh, acc_out, cid, sid):
    off = pl.multiple_of(sid * _ZCHUNK, 8)
    pltpu.sync_copy(acc_sh.at[pl.ds(off, _ZCHUNK)],
                    acc_out.at[cid].at[pl.ds(off, _ZCHUNK)])

    @pl.when(sid == NUM_SUBCORES - 1)
    def _():
        pltpu.sync_copy(acc_sh.at[pl.ds(NUM_SUBCORES * _ZCHUNK, _ZTAIL)],
                        acc_out.at[cid].at[pl.ds(NUM_SUBCORES * _ZCHUNK, _ZTAIL)])


DEG_W = 16


def _sc_degree(dstg_hbm, zeros_hbm, ones_hbm, deg_out, deg_sh, dst_v, ones_v,
               *sems):
    cid = lax.axis_index("c")
    sid = lax.axis_index("s")
    w = cid * NUM_SUBCORES + sid

    _zero_init(zeros_hbm, deg_sh, sid)
    pltpu.sync_copy(ones_hbm, ones_v)
    pltpu.sync_copy(dstg_hbm.at[w], dst_v)
    plsc.subcore_barrier()

    for b in range(PIPE_N):
        pltpu.async_copy(ones_v, deg_sh.at[dst_v.at[b]], sems[b], add=True)

    def round_(g, _):
        for b in range(PIPE_N):
            j = g * PIPE_N + b
            pltpu.make_async_copy(ones_v, deg_sh.at[dst_v.at[j]],
                                  sems[b]).wait()
            pltpu.async_copy(ones_v, deg_sh.at[dst_v.at[j]], sems[b],
                             add=True)
        return 0

    lax.fori_loop(1, NB_N // PIPE_N, round_, 0)
    for b in range(PIPE_N):
        j = NB_N - PIPE_N + b
        pltpu.make_async_copy(ones_v, deg_sh.at[dst_v.at[j]], sems[b]).wait()
    plsc.subcore_barrier()
    _copy_out(deg_sh, deg_out, cid, sid)


def _degree_parts(dstg, zeros_nw, ones_bw):
    return pl.kernel(
        _sc_degree,
        out_type=jax.ShapeDtypeStruct((NUM_CORES, N_NODES, DEG_W), jnp.float32),
        mesh=_SC_MESH,
        scratch_types=[
            pltpu.VMEM_SHARED((N_NODES, DEG_W), jnp.float32),
            pltpu.VMEM((NB_N, BATCH_N), jnp.int32),
            pltpu.VMEM((BATCH_N, DEG_W), jnp.float32),
        ] + [pltpu.SemaphoreType.DMA] * PIPE_N,
        compiler_params=pltpu.CompilerParams(use_tc_tiling_on_sc=False),
    )(dstg, zeros_nw, ones_bw)


def _sc_aggregate(d, batch, nb, pipe, tab_hbm, srcg_hbm, dstg_hbm, zeros_hbm,
                  acc_out, acc_sh, src_v, dst_v, rows_v, *sems):
    cid = lax.axis_index("c")
    sid = lax.axis_index("s")
    w = cid * NUM_SUBCORES + sid
    gsems = sems[:pipe]
    ssems = sems[pipe:2 * pipe]
    zc, zt, si, di = sems[2 * pipe:]

    off = pl.multiple_of(sid * _ZCHUNK, 8)
    zd = pltpu.async_copy(zeros_hbm.at[pl.ds(off, _ZCHUNK)],
                          acc_sh.at[pl.ds(off, _ZCHUNK)], zc)

    @pl.when(sid == NUM_SUBCORES - 1)
    def _():
        pltpu.async_copy(zeros_hbm.at[pl.ds(NUM_SUBCORES * _ZCHUNK, _ZTAIL)],
                         acc_sh.at[pl.ds(NUM_SUBCORES * _ZCHUNK, _ZTAIL)], zt)

    sd_src = pltpu.async_copy(srcg_hbm.at[w], src_v, si)
    sd_dst = pltpu.async_copy(dstg_hbm.at[w], dst_v, di)
    sd_src.wait()

    for b in range(pipe):
        pltpu.async_copy(tab_hbm.at[src_v.at[b]], rows_v.at[b], gsems[b])

    zd.wait()

    @pl.when(sid == NUM_SUBCORES - 1)
    def _():
        pltpu.make_async_copy(
            zeros_hbm.at[pl.ds(NUM_SUBCORES * _ZCHUNK, _ZTAIL)],
            acc_sh.at[pl.ds(NUM_SUBCORES * _ZCHUNK, _ZTAIL)], zt).wait()

    sd_dst.wait()
    plsc.subcore_barrier()

    rounds = nb // pipe

    def round_(g, _):
        sd = []
        for b in range(pipe):
            j = g * pipe + b
            pltpu.make_async_copy(tab_hbm.at[src_v.at[j]], rows_v.at[b],
                                  gsems[b]).wait()
            sd.append(pltpu.async_copy(
                rows_v.at[b], acc_sh.at[dst_v.at[j]], ssems[b], add=True))
        for b in range(pipe):
            sd[b].wait()

            @pl.when(g < rounds - 1)
            def _(b=b):
                jn = (g + 1) * pipe + b
                pltpu.async_copy(tab_hbm.at[src_v.at[jn]], rows_v.at[b],
                                 gsems[b])
        return 0

    lax.fori_loop(0, rounds, round_, 0)
    plsc.subcore_barrier()
    _copy_out(acc_sh, acc_out, cid, sid)


def _aggregate_parts(d, batch, nb, pipe, tab, srcg, dstg, zeros_nd):
    return pl.kernel(
        functools.partial(_sc_aggregate, d, batch, nb, pipe),
        out_type=jax.ShapeDtypeStruct((NUM_CORES, N_NODES, d), jnp.float32),
        mesh=_SC_MESH,
        scratch_types=[
            pltpu.VMEM_SHARED((N_NODES, d), jnp.float32),
            pltpu.VMEM((nb, batch), jnp.int32),
            pltpu.VMEM((nb, batch), jnp.int32),
            pltpu.VMEM((pipe, batch, d), jnp.float32),
        ] + [pltpu.SemaphoreType.DMA] * (2 * pipe + 4),
        compiler_params=pltpu.CompilerParams(use_tc_tiling_on_sc=False),
    )(tab, srcg, dstg, zeros_nd)


def _tc_mm1_scale(x_ref, w1_ref, degp_ref, xws_ref, dinv_ref):
    deg = 1.0 + degp_ref[0, :, 0] + degp_ref[1, :, 0]
    dinv = lax.rsqrt(deg)
    dinv_ref[:, 0] = dinv
    xws_ref[...] = (x_ref[...] @ w1_ref[...]) * dinv[:, None]


def _tc_middle(accp_ref, xws_ref, dinv_ref, b1_ref, w2_ref, out_ref):
    dinv = dinv_ref[:, 0][:, None]
    h = dinv * (accp_ref[0] + accp_ref[1] + xws_ref[...]) + b1_ref[...]
    h = jnp.maximum(h, 0.0)
    out_ref[...] = (h @ w2_ref[...]) * dinv


def _tc_final(accp_ref, hw2s_ref, dinv_ref, b2_ref, out_ref):
    dinv = dinv_ref[:, 0][:, None]
    logits = dinv * (accp_ref[0] + accp_ref[1] + hw2s_ref[...]) + b2_ref[...]
    m = jnp.max(logits, axis=1, keepdims=True)
    e = jnp.exp(logits - m)
    out_ref[...] = e / jnp.sum(e, axis=1, keepdims=True)


_ROW_BLK = 2000
_N_BLKS = N_NODES // _ROW_BLK


def _mm1_scale(x, w1, degp):
    return pl.pallas_call(
        _tc_mm1_scale,
        grid=(_N_BLKS,),
        in_specs=[
            pl.BlockSpec((_ROW_BLK, D_IN), lambda i: (i, 0)),
            pl.BlockSpec((D_IN, D_HID), lambda i: (0, 0)),
            pl.BlockSpec((NUM_CORES, _ROW_BLK, DEG_W), lambda i: (0, i, 0)),
        ],
        out_specs=[
            pl.BlockSpec((_ROW_BLK, D_HID), lambda i: (i, 0)),
            pl.BlockSpec((_ROW_BLK, 1), lambda i: (i, 0)),
        ],
        out_shape=[
            jax.ShapeDtypeStruct((N_NODES, D_HID), jnp.float32),
            jax.ShapeDtypeStruct((N_NODES, 1), jnp.float32),
        ],
    )(x, w1, degp)


def _middle(accp, xws, dinv, b1, w2):
    return pl.pallas_call(
        _tc_middle,
        grid=(_N_BLKS,),
        in_specs=[
            pl.BlockSpec((NUM_CORES, _ROW_BLK, D_HID), lambda i: (0, i, 0)),
            pl.BlockSpec((_ROW_BLK, D_HID), lambda i: (i, 0)),
            pl.BlockSpec((_ROW_BLK, 1), lambda i: (i, 0)),
            pl.BlockSpec((1, D_HID), lambda i: (0, 0)),
            pl.BlockSpec((D_HID, N_CLS), lambda i: (0, 0)),
        ],
        out_specs=pl.BlockSpec((_ROW_BLK, N_CLS), lambda i: (i, 0)),
        out_shape=jax.ShapeDtypeStruct((N_NODES, N_CLS), jnp.float32),
    )(accp, xws, dinv, b1, w2)


def _final(accp, hw2s, dinv, b2):
    return pl.pallas_call(
        _tc_final,
        grid=(_N_BLKS,),
        in_specs=[
            pl.BlockSpec((NUM_CORES, _ROW_BLK, N_CLS), lambda i: (0, i, 0)),
            pl.BlockSpec((_ROW_BLK, N_CLS), lambda i: (i, 0)),
            pl.BlockSpec((_ROW_BLK, 1), lambda i: (i, 0)),
            pl.BlockSpec((1, N_CLS), lambda i: (0, 0)),
        ],
        out_specs=pl.BlockSpec((_ROW_BLK, N_CLS), lambda i: (i, 0)),
        out_shape=jax.ShapeDtypeStruct((N_NODES, N_CLS), jnp.float32),
    )(accp, hw2s, dinv, b2)


@jax.jit
def kernel(node_features, edge_index, W1, b1, W2, b2):
    grp_w = (NUM_WORKERS, NB_W, BATCH_W)
    grp_n = (NUM_WORKERS, NB_N, BATCH_N)
    srcg_w = edge_index[0].reshape(grp_w)
    dstg_w = edge_index[1].reshape(grp_w)
    srcg_n = edge_index[0].reshape(grp_n)
    dstg_n = edge_index[1].reshape(grp_n)
    zeros_nw = jnp.zeros((N_NODES, DEG_W), jnp.float32)
    ones_bw = jnp.ones((BATCH_N, DEG_W), jnp.float32)
    zeros_nh = jnp.zeros((N_NODES, D_HID), jnp.float32)
    zeros_nc = jnp.zeros((N_NODES, N_CLS), jnp.float32)

    degp = _degree_parts(dstg_n, zeros_nw, ones_bw)
    xws, dinv = _mm1_scale(node_features, W1, degp)
    accp1 = _aggregate_parts(D_HID, BATCH_W, NB_W, PIPE_W, xws, srcg_w,
                             dstg_w, zeros_nh)
    hw2s = _middle(accp1, xws, dinv, b1.reshape(1, D_HID), W2)
    accp2 = _aggregate_parts(N_CLS, BATCH_N, NB_N, PIPE_N, hw2s, srcg_n,
                             dstg_n, zeros_nc)
    return _final(accp2, hw2s, dinv, b2.reshape(1, N_CLS))


# speedup vs baseline: 46.5287x; 1.1090x over previous
import functools
import jax
import jax.numpy as jnp
from jax import lax
from jax.experimental import pallas as pl
from jax.experimental.pallas import tpu as pltpu
from jax.experimental.pallas import tpu_sc as plsc

N_NODES = 10000
N_EDGES = 320000
D_IN = 128
D_HID = 128
N_CLS = 16

NUM_CORES = 2
NUM_SUBCORES = 16
NUM_WORKERS = NUM_CORES * NUM_SUBCORES
EDGES_PER_CORE = N_EDGES // NUM_CORES
EDGES_PER_WORKER = EDGES_PER_CORE // NUM_SUBCORES
PIPE_W = 5
PIPE_N = 10

BATCH_W = 100
NB_W = EDGES_PER_WORKER // BATCH_W
BATCH_N = 100
NB_N = EDGES_PER_WORKER // BATCH_N

_SC_MESH = plsc.VectorSubcoreMesh(
    core_axis_name="c", subcore_axis_name="s",
    num_cores=NUM_CORES, num_subcores=NUM_SUBCORES)

_ZCHUNK = 624
_ZTAIL = N_NODES - NUM_SUBCORES * _ZCHUNK


def _zero_init(zeros_hbm, acc_sh, sid):
    off = pl.multiple_of(sid * _ZCHUNK, 8)
    pltpu.sync_copy(zeros_hbm.at[pl.ds(off, _ZCHUNK)],
                    acc_sh.at[pl.ds(off, _ZCHUNK)])

    @pl.when(sid == NUM_SUBCORES - 1)
    def _():
        pltpu.sync_copy(zeros_hbm.at[pl.ds(NUM_SUBCORES * _ZCHUNK, _ZTAIL)],
                        acc_sh.at[pl.ds(NUM_SUBCORES * _ZCHUNK, _ZTAIL)])


def _copy_out(acc_sh, acc_out, cid, sid):
    off = pl.multiple_of(sid * _ZCHUNK, 8)
    pltpu.sync_copy(acc_sh.at[pl.ds(off, _ZCHUNK)],
                    acc_out.at[cid].at[pl.ds(off, _ZCHUNK)])

    @pl.when(sid == NUM_SUBCORES - 1)
    def _():
        pltpu.sync_copy(acc_sh.at[pl.ds(NUM_SUBCORES * _ZCHUNK, _ZTAIL)],
                        acc_out.at[cid].at[pl.ds(NUM_SUBCORES * _ZCHUNK, _ZTAIL)])


DEG_W = 16


def _sc_degree(dstg_hbm, zeros_hbm, ones_hbm, deg_out, deg_sh, dst_v, ones_v,
               *sems):
    cid = lax.axis_index("c")
    sid = lax.axis_index("s")
    w = cid * NUM_SUBCORES + sid

    _zero_init(zeros_hbm, deg_sh, sid)
    pltpu.sync_copy(ones_hbm, ones_v)
    pltpu.sync_copy(dstg_hbm.at[w], dst_v)
    plsc.subcore_barrier()

    for b in range(PIPE_N):
        pltpu.async_copy(ones_v, deg_sh.at[dst_v.at[b]], sems[b], add=True)

    def round_(g, _):
        for b in range(PIPE_N):
            j = g * PIPE_N + b
            pltpu.make_async_copy(ones_v, deg_sh.at[dst_v.at[j]],
                                  sems[b]).wait()
            pltpu.async_copy(ones_v, deg_sh.at[dst_v.at[j]], sems[b],
                             add=True)
        return 0

    lax.fori_loop(1, NB_N // PIPE_N, round_, 0)
    for b in range(PIPE_N):
        j = NB_N - PIPE_N + b
        pltpu.make_async_copy(ones_v, deg_sh.at[dst_v.at[j]], sems[b]).wait()
    plsc.subcore_barrier()
    _copy_out(deg_sh, deg_out, cid, sid)


def _degree_parts(dstg, zeros_nw, ones_bw):
    return pl.kernel(
        _sc_degree,
        out_type=jax.ShapeDtypeStruct((NUM_CORES, N_NODES, DEG_W), jnp.float32),
        mesh=_SC_MESH,
        scratch_types=[
            pltpu.VMEM_SHARED((N_NODES, DEG_W), jnp.float32),
            pltpu.VMEM((NB_N, BATCH_N), jnp.int32),
            pltpu.VMEM((BATCH_N, DEG_W), jnp.float32),
        ] + [pltpu.SemaphoreType.DMA] * PIPE_N,
        compiler_params=pltpu.CompilerParams(use_tc_tiling_on_sc=False),
    )(dstg, zeros_nw, ones_bw)


def _sc_aggregate(d, batch, nb, pipe, tab_hbm, srcg_hbm, dstg_hbm, zeros_hbm,
                  acc_out, acc_sh, src_v, dst_v, rows_v, *sems):
    cid = lax.axis_index("c")
    sid = lax.axis_index("s")
    w = cid * NUM_SUBCORES + sid
    gsems = sems[:pipe]
    ssems = sems[pipe:2 * pipe]
    zc, zt, si, di = sems[2 * pipe:]

    off = pl.multiple_of(sid * _ZCHUNK, 8)
    zd = pltpu.async_copy(zeros_hbm.at[pl.ds(off, _ZCHUNK)],
                          acc_sh.at[pl.ds(off, _ZCHUNK)], zc)

    @pl.when(sid == NUM_SUBCORES - 1)
    def _():
        pltpu.async_copy(zeros_hbm.at[pl.ds(NUM_SUBCORES * _ZCHUNK, _ZTAIL)],
                         acc_sh.at[pl.ds(NUM_SUBCORES * _ZCHUNK, _ZTAIL)], zt)

    sd_src = pltpu.async_copy(srcg_hbm.at[w], src_v, si)
    sd_dst = pltpu.async_copy(dstg_hbm.at[w], dst_v, di)
    sd_src.wait()

    for b in range(pipe):
        pltpu.async_copy(tab_hbm.at[src_v.at[b]], rows_v.at[b], gsems[b])

    zd.wait()

    @pl.when(sid == NUM_SUBCORES - 1)
    def _():
        pltpu.make_async_copy(
            zeros_hbm.at[pl.ds(NUM_SUBCORES * _ZCHUNK, _ZTAIL)],
            acc_sh.at[pl.ds(NUM_SUBCORES * _ZCHUNK, _ZTAIL)], zt).wait()

    sd_dst.wait()
    plsc.subcore_barrier()

    rounds = nb // pipe

    def round_(g, _):
        sd = []
        for b in range(pipe):
            j = g * pipe + b
            pltpu.make_async_copy(tab_hbm.at[src_v.at[j]], rows_v.at[b],
                                  gsems[b]).wait()
            sd.append(pltpu.async_copy(
                rows_v.at[b], acc_sh.at[dst_v.at[j]], ssems[b], add=True))
        for b in range(pipe):
            sd[b].wait()

            @pl.when(g < rounds - 1)
            def _(b=b):
                jn = (g + 1) * pipe + b
                pltpu.async_copy(tab_hbm.at[src_v.at[jn]], rows_v.at[b],
                                 gsems[b])
        return 0

    lax.fori_loop(0, rounds, round_, 0)
    plsc.subcore_barrier()
    _copy_out(acc_sh, acc_out, cid, sid)


def _aggregate_parts(d, batch, nb, pipe, tab, srcg, dstg, zeros_nd):
    dt = tab.dtype
    return pl.kernel(
        functools.partial(_sc_aggregate, d, batch, nb, pipe),
        out_type=jax.ShapeDtypeStruct((NUM_CORES, N_NODES, d), dt),
        mesh=_SC_MESH,
        scratch_types=[
            pltpu.VMEM_SHARED((N_NODES, d), dt),
            pltpu.VMEM((nb, batch), jnp.int32),
            pltpu.VMEM((nb, batch), jnp.int32),
            pltpu.VMEM((pipe, batch, d), dt),
        ] + [pltpu.SemaphoreType.DMA] * (2 * pipe + 4),
        compiler_params=pltpu.CompilerParams(use_tc_tiling_on_sc=False),
    )(tab, srcg, dstg, zeros_nd)


def _tc_mm1_scale(x_ref, w1_ref, degp_ref, xws_ref, dinv_ref):
    deg = 1.0 + degp_ref[0, :, 0] + degp_ref[1, :, 0]
    dinv = lax.rsqrt(deg)
    dinv_ref[:, 0] = dinv
    xws_ref[...] = ((x_ref[...] @ w1_ref[...]) * dinv[:, None]).astype(
        jnp.bfloat16)


def _tc_middle(accp_ref, xws_ref, dinv_ref, b1_ref, w2_ref, out_ref):
    dinv = dinv_ref[:, 0][:, None]
    agg = (accp_ref[0].astype(jnp.float32) + accp_ref[1].astype(jnp.float32)
           + xws_ref[...].astype(jnp.float32))
    h = dinv * agg + b1_ref[...]
    h = jnp.maximum(h, 0.0)
    out_ref[...] = (h @ w2_ref[...]) * dinv


def _tc_final(accp_ref, hw2s_ref, dinv_ref, b2_ref, out_ref):
    dinv = dinv_ref[:, 0][:, None]
    logits = dinv * (accp_ref[0] + accp_ref[1] + hw2s_ref[...]) + b2_ref[...]
    m = jnp.max(logits, axis=1, keepdims=True)
    e = jnp.exp(logits - m)
    out_ref[...] = e / jnp.sum(e, axis=1, keepdims=True)


_ROW_BLK = 2000
_N_BLKS = N_NODES // _ROW_BLK


def _mm1_scale(x, w1, degp):
    return pl.pallas_call(
        _tc_mm1_scale,
        grid=(_N_BLKS,),
        in_specs=[
            pl.BlockSpec((_ROW_BLK, D_IN), lambda i: (i, 0)),
            pl.BlockSpec((D_IN, D_HID), lambda i: (0, 0)),
            pl.BlockSpec((NUM_CORES, _ROW_BLK, DEG_W), lambda i: (0, i, 0)),
        ],
        out_specs=[
            pl.BlockSpec((_ROW_BLK, D_HID), lambda i: (i, 0)),
            pl.BlockSpec((_ROW_BLK, 1), lambda i: (i, 0)),
        ],
        out_shape=[
            jax.ShapeDtypeStruct((N_NODES, D_HID), jnp.bfloat16),
            jax.ShapeDtypeStruct((N_NODES, 1), jnp.float32),
        ],
    )(x, w1, degp)


def _middle(accp, xws, dinv, b1, w2):
    return pl.pallas_call(
        _tc_middle,
        grid=(_N_BLKS,),
        in_specs=[
            pl.BlockSpec((NUM_CORES, _ROW_BLK, D_HID), lambda i: (0, i, 0)),
            pl.BlockSpec((_ROW_BLK, D_HID), lambda i: (i, 0)),
            pl.BlockSpec((_ROW_BLK, 1), lambda i: (i, 0)),
            pl.BlockSpec((1, D_HID), lambda i: (0, 0)),
            pl.BlockSpec((D_HID, N_CLS), lambda i: (0, 0)),
        ],
        out_specs=pl.BlockSpec((_ROW_BLK, N_CLS), lambda i: (i, 0)),
        out_shape=jax.ShapeDtypeStruct((N_NODES, N_CLS), jnp.float32),
    )(accp, xws, dinv, b1, w2)


def _final(accp, hw2s, dinv, b2):
    return pl.pallas_call(
        _tc_final,
        grid=(_N_BLKS,),
        in_specs=[
            pl.BlockSpec((NUM_CORES, _ROW_BLK, N_CLS), lambda i: (0, i, 0)),
            pl.BlockSpec((_ROW_BLK, N_CLS), lambda i: (i, 0)),
            pl.BlockSpec((_ROW_BLK, 1), lambda i: (i, 0)),
            pl.BlockSpec((1, N_CLS), lambda i: (0, 0)),
        ],
        out_specs=pl.BlockSpec((_ROW_BLK, N_CLS), lambda i: (i, 0)),
        out_shape=jax.ShapeDtypeStruct((N_NODES, N_CLS), jnp.float32),
    )(accp, hw2s, dinv, b2)


@jax.jit
def kernel(node_features, edge_index, W1, b1, W2, b2):
    grp_w = (NUM_WORKERS, NB_W, BATCH_W)
    grp_n = (NUM_WORKERS, NB_N, BATCH_N)
    srcg_w = edge_index[0].reshape(grp_w)
    dstg_w = edge_index[1].reshape(grp_w)
    srcg_n = edge_index[0].reshape(grp_n)
    dstg_n = edge_index[1].reshape(grp_n)
    zeros_nw = jnp.zeros((N_NODES, DEG_W), jnp.float32)
    ones_bw = jnp.ones((BATCH_N, DEG_W), jnp.float32)
    zeros_nh = jnp.zeros((N_NODES, D_HID), jnp.bfloat16)
    zeros_nc = jnp.zeros((N_NODES, N_CLS), jnp.float32)

    degp = _degree_parts(dstg_n, zeros_nw, ones_bw)
    xws, dinv = _mm1_scale(node_features, W1, degp)
    accp1 = _aggregate_parts(D_HID, BATCH_W, NB_W, PIPE_W, xws, srcg_w,
                             dstg_w, zeros_nh)
    hw2s = _middle(accp1, xws, dinv, b1.reshape(1, D_HID), W2)
    accp2 = _aggregate_parts(N_CLS, BATCH_N, NB_N, PIPE_N, hw2s, srcg_n,
                             dstg_n, zeros_nc)
    return _final(accp2, hw2s, dinv, b2.reshape(1, N_CLS))
